# baseline (device time: 31511 ns/iter reference)
import jax
import jax.numpy as jnp
from jax import lax
from jax.experimental import pallas as pl
from jax.experimental.pallas import tpu as pltpu

N_DEV = 4
HQ_LOC = 4
HQ = 16
DH = 64
SQ_LOC = 256
B = 2
GP = 2
TILE = 2 * DH
D_MODEL = 512
ROWS = B * SQ_LOC
HID_LOC = HQ_LOC * DH
PAY_ROWS = 2 * D_MODEL
HALF = D_MODEL


def kernel(x, Wq, K_ext, V_ext, Wo):
    x2 = x.reshape(ROWS, D_MODEL)

    def body(x_ref, wq_ref, wo_ref, k_ref, v_ref, out_ref,
             comm_a_ref, comm_b_ref, pay_a_ref, pay_b_ref,
             ctx_ref, xbf_ref, kbf_ref, vbf_ref, acc_ref,
             send_sems, recv_sems):
        my = lax.axis_index("i")
        left = lax.rem(my + (N_DEV - 1), N_DEV)
        right = lax.rem(my + 1, N_DEV)

        pay_a_ref[...] = (wq_ref[...] * 0.125).astype(jnp.bfloat16)
        pay_b_ref[...] = wo_ref[...].astype(jnp.bfloat16)

        barrier = pltpu.get_barrier_semaphore()
        for nbr in (left, right):
            pl.semaphore_signal(
                barrier, inc=1,
                device_id=(nbr,), device_id_type=pl.DeviceIdType.MESH,
            )
        pl.semaphore_wait(barrier, 2)

        def half_send(src, comm, dst_slot, sem_i, dev):
            return pltpu.make_async_remote_copy(
                src_ref=src, dst_ref=comm.at[dst_slot],
                send_sem=send_sems.at[sem_i], recv_sem=recv_sems.at[sem_i],
                device_id=(dev,), device_id_type=pl.DeviceIdType.MESH,
            )

        send_a_right = half_send(pay_a_ref, comm_a_ref, 0, 0, right)
        send_a_left = half_send(pay_a_ref, comm_a_ref, 1, 1, left)
        send_b_right = half_send(pay_b_ref, comm_b_ref, 0, 2, right)
        send_b_left = half_send(pay_b_ref, comm_b_ref, 1, 3, left)
        fwd_a_right = pltpu.make_async_remote_copy(
            src_ref=comm_a_ref.at[0], dst_ref=comm_a_ref.at[2],
            send_sem=send_sems.at[4], recv_sem=recv_sems.at[4],
            device_id=(right,), device_id_type=pl.DeviceIdType.MESH,
        )
        fwd_b_left = pltpu.make_async_remote_copy(
            src_ref=comm_b_ref.at[1], dst_ref=comm_b_ref.at[2],
            send_sem=send_sems.at[5], recv_sem=recv_sems.at[5],
            device_id=(left,), device_id_type=pl.DeviceIdType.MESH,
        )

        send_a_right.start()
        send_a_left.start()
        send_b_right.start()
        send_b_left.start()

        xbf_ref[...] = x_ref[...].astype(jnp.bfloat16)
        for h in range(HQ):
            for b in range(B):
                kbf_ref[h, b] = k_ref[b, :, h, :].astype(jnp.bfloat16)
                vbf_ref[h, b] = v_ref[b, :, h, :].astype(jnp.bfloat16)

        row = lax.broadcasted_iota(jnp.int32, (TILE, TILE), 0)
        col = lax.broadcasted_iota(jnp.int32, (TILE, TILE), 1)
        maskf = ((row // DH) == (col // DH)).astype(jnp.float32)

        def attn(wq_c, origin):
            q_bf = jnp.dot(
                xbf_ref[...], wq_c, preferred_element_type=jnp.float32
            ).astype(jnp.bfloat16)
            for b in range(B):
                for g in range(GP):
                    r0 = b * SQ_LOC + g * TILE
                    for t in range(HQ_LOC):
                        head = origin * HQ_LOC + t
                        q = q_bf[r0:r0 + TILE, t * DH:(t + 1) * DH]
                        k = kbf_ref[head, b, g * TILE:(g + 1) * TILE, :]
                        s = lax.dot_general(
                            q, k, (((1,), (1,)), ((), ())),
                            preferred_element_type=jnp.float32,
                        )
                        e = jnp.exp(s) * maskf
                        denom = jnp.sum(e, axis=1, keepdims=True)
                        v = vbf_ref[head, b, g * TILE:(g + 1) * TILE, :]
                        ctx = jnp.dot(
                            e.astype(jnp.bfloat16), v,
                            preferred_element_type=jnp.float32,
                        ) * (1.0 / denom)
                        ctx_ref[r0:r0 + TILE, t * DH:(t + 1) * DH] = (
                            ctx.astype(jnp.bfloat16)
                        )

        def proj(wo_c, first):
            contrib = lax.dot_general(
                ctx_ref[...], wo_c, (((1,), (0,)), ((), ())),
                preferred_element_type=jnp.float32,
            )
            if first:
                acc_ref[...] = contrib
            else:
                acc_ref[...] += contrib

        attn(pay_a_ref[...], my)
        proj(pay_b_ref[...], True)

        send_a_right.wait_recv()
        fwd_a_right.start()
        attn(comm_a_ref[0], lax.rem(my + (N_DEV - 1), N_DEV))
        send_b_right.wait_recv()
        proj(comm_b_ref[0], False)
        fwd_a_right.wait_send()

        send_a_left.wait_recv()
        attn(comm_a_ref[1], lax.rem(my + 1, N_DEV))
        send_b_left.wait_recv()
        fwd_b_left.start()
        proj(comm_b_ref[1], False)

        fwd_a_right.wait_recv()
        attn(comm_a_ref[2], lax.rem(my + 2, N_DEV))
        fwd_b_left.wait_recv()
        proj(comm_b_ref[2], False)

        out_ref[...] = acc_ref[...].astype(jnp.bfloat16)

        send_a_right.wait_send()
        send_a_left.wait_send()
        send_b_right.wait_send()
        send_b_left.wait_send()
        fwd_b_left.wait_send()

    out = pl.pallas_call(
        body,
        out_shape=jax.ShapeDtypeStruct((ROWS, D_MODEL), jnp.bfloat16),
        in_specs=[
            pl.BlockSpec(memory_space=pltpu.VMEM),
            pl.BlockSpec(memory_space=pltpu.VMEM),
            pl.BlockSpec(memory_space=pltpu.VMEM),
            pl.BlockSpec(memory_space=pltpu.VMEM),
            pl.BlockSpec(memory_space=pltpu.VMEM),
        ],
        out_specs=pl.BlockSpec(memory_space=pltpu.VMEM),
        scratch_shapes=[
            pltpu.VMEM((3, D_MODEL, HID_LOC), jnp.bfloat16),
            pltpu.VMEM((3, HID_LOC, D_MODEL), jnp.bfloat16),
            pltpu.VMEM((D_MODEL, HID_LOC), jnp.bfloat16),
            pltpu.VMEM((HID_LOC, D_MODEL), jnp.bfloat16),
            pltpu.VMEM((ROWS, HID_LOC), jnp.bfloat16),
            pltpu.VMEM((ROWS, D_MODEL), jnp.bfloat16),
            pltpu.VMEM((HQ, B, SQ_LOC, DH), jnp.bfloat16),
            pltpu.VMEM((HQ, B, SQ_LOC, DH), jnp.bfloat16),
            pltpu.VMEM((ROWS, D_MODEL), jnp.float32),
            pltpu.SemaphoreType.DMA((6,)),
            pltpu.SemaphoreType.DMA((6,)),
        ],
        compiler_params=pltpu.CompilerParams(collective_id=0),
    )(x2, Wq, Wo, K_ext, V_ext)

    return out.reshape(B, SQ_LOC, D_MODEL)


# device time: 25837 ns/iter; 1.2196x vs baseline; 1.2196x over previous
import jax
import jax.numpy as jnp
from jax import lax
from jax.experimental import pallas as pl
from jax.experimental.pallas import tpu as pltpu

N_DEV = 4
HQ_LOC = 4
HQ = 16
DH = 64
SQ_LOC = 256
B = 2
GP = 2
TILE = 2 * DH
D_MODEL = 512
ROWS = B * SQ_LOC
HID_LOC = HQ_LOC * DH
PAY_ROWS = 2 * D_MODEL
HALF = D_MODEL


def kernel(x, Wq, K_ext, V_ext, Wo):
    x2 = x.reshape(ROWS, D_MODEL)
    kb = jnp.transpose(K_ext, (2, 0, 1, 3)).astype(jnp.bfloat16)
    vb = jnp.transpose(V_ext, (2, 0, 1, 3)).astype(jnp.bfloat16)

    def body(x_ref, wq_ref, wo_ref, kbf_ref, vbf_ref, out_ref,
             comm_a_ref, comm_b_ref, pay_a_ref, pay_b_ref,
             ctx_ref, xbf_ref, acc_ref,
             send_sems, recv_sems):
        my = lax.axis_index("i")
        left = lax.rem(my + (N_DEV - 1), N_DEV)
        right = lax.rem(my + 1, N_DEV)

        pay_a_ref[...] = (wq_ref[...] * 0.125).astype(jnp.bfloat16)
        pay_b_ref[...] = wo_ref[...].astype(jnp.bfloat16)

        barrier = pltpu.get_barrier_semaphore()
        for nbr in (left, right):
            pl.semaphore_signal(
                barrier, inc=1,
                device_id=(nbr,), device_id_type=pl.DeviceIdType.MESH,
            )
        pl.semaphore_wait(barrier, 2)

        def half_send(src, comm, dst_slot, sem_i, dev):
            return pltpu.make_async_remote_copy(
                src_ref=src, dst_ref=comm.at[dst_slot],
                send_sem=send_sems.at[sem_i], recv_sem=recv_sems.at[sem_i],
                device_id=(dev,), device_id_type=pl.DeviceIdType.MESH,
            )

        send_a_right = half_send(pay_a_ref, comm_a_ref, 0, 0, right)
        send_a_left = half_send(pay_a_ref, comm_a_ref, 1, 1, left)
        send_b_right = half_send(pay_b_ref, comm_b_ref, 0, 2, right)
        send_b_left = half_send(pay_b_ref, comm_b_ref, 1, 3, left)
        fwd_a_right = pltpu.make_async_remote_copy(
            src_ref=comm_a_ref.at[0], dst_ref=comm_a_ref.at[2],
            send_sem=send_sems.at[4], recv_sem=recv_sems.at[4],
            device_id=(right,), device_id_type=pl.DeviceIdType.MESH,
        )
        fwd_b_left = pltpu.make_async_remote_copy(
            src_ref=comm_b_ref.at[1], dst_ref=comm_b_ref.at[2],
            send_sem=send_sems.at[5], recv_sem=recv_sems.at[5],
            device_id=(left,), device_id_type=pl.DeviceIdType.MESH,
        )

        send_a_right.start()
        send_a_left.start()
        send_b_right.start()
        send_b_left.start()

        xbf_ref[...] = x_ref[...].astype(jnp.bfloat16)

        row = lax.broadcasted_iota(jnp.int32, (TILE, TILE), 0)
        col = lax.broadcasted_iota(jnp.int32, (TILE, TILE), 1)
        maskf = ((row // DH) == (col // DH)).astype(jnp.float32)

        def attn(wq_c, origin):
            q_bf = jnp.dot(
                xbf_ref[...], wq_c, preferred_element_type=jnp.float32
            ).astype(jnp.bfloat16)
            for b in range(B):
                for g in range(GP):
                    r0 = b * SQ_LOC + g * TILE
                    for t in range(HQ_LOC):
                        head = origin * HQ_LOC + t
                        q = q_bf[r0:r0 + TILE, t * DH:(t + 1) * DH]
                        k = kbf_ref[head, b, g * TILE:(g + 1) * TILE, :]
                        s = lax.dot_general(
                            q, k, (((1,), (1,)), ((), ())),
                            preferred_element_type=jnp.float32,
                        )
                        e = jnp.exp(s) * maskf
                        denom = jnp.sum(e, axis=1, keepdims=True)
                        v = vbf_ref[head, b, g * TILE:(g + 1) * TILE, :]
                        ctx = jnp.dot(
                            e.astype(jnp.bfloat16), v,
                            preferred_element_type=jnp.float32,
                        ) * (1.0 / denom)
                        ctx_ref[r0:r0 + TILE, t * DH:(t + 1) * DH] = (
                            ctx.astype(jnp.bfloat16)
                        )

        def proj(wo_c, first):
            contrib = lax.dot_general(
                ctx_ref[...], wo_c, (((1,), (0,)), ((), ())),
                preferred_element_type=jnp.float32,
            )
            if first:
                acc_ref[...] = contrib
            else:
                acc_ref[...] += contrib

        attn(pay_a_ref[...], my)
        proj(pay_b_ref[...], True)

        send_a_right.wait_recv()
        fwd_a_right.start()
        attn(comm_a_ref[0], lax.rem(my + (N_DEV - 1), N_DEV))
        send_b_right.wait_recv()
        proj(comm_b_ref[0], False)
        fwd_a_right.wait_send()

        send_a_left.wait_recv()
        attn(comm_a_ref[1], lax.rem(my + 1, N_DEV))
        send_b_left.wait_recv()
        fwd_b_left.start()
        proj(comm_b_ref[1], False)

        fwd_a_right.wait_recv()
        attn(comm_a_ref[2], lax.rem(my + 2, N_DEV))
        fwd_b_left.wait_recv()
        proj(comm_b_ref[2], False)

        out_ref[...] = acc_ref[...].astype(jnp.bfloat16)

        send_a_right.wait_send()
        send_a_left.wait_send()
        send_b_right.wait_send()
        send_b_left.wait_send()
        fwd_b_left.wait_send()

    out = pl.pallas_call(
        body,
        out_shape=jax.ShapeDtypeStruct((ROWS, D_MODEL), jnp.bfloat16),
        in_specs=[
            pl.BlockSpec(memory_space=pltpu.VMEM),
            pl.BlockSpec(memory_space=pltpu.VMEM),
            pl.BlockSpec(memory_space=pltpu.VMEM),
            pl.BlockSpec(memory_space=pltpu.VMEM),
            pl.BlockSpec(memory_space=pltpu.VMEM),
        ],
        out_specs=pl.BlockSpec(memory_space=pltpu.VMEM),
        scratch_shapes=[
            pltpu.VMEM((3, D_MODEL, HID_LOC), jnp.bfloat16),
            pltpu.VMEM((3, HID_LOC, D_MODEL), jnp.bfloat16),
            pltpu.VMEM((D_MODEL, HID_LOC), jnp.bfloat16),
            pltpu.VMEM((HID_LOC, D_MODEL), jnp.bfloat16),
            pltpu.VMEM((ROWS, HID_LOC), jnp.bfloat16),
            pltpu.VMEM((ROWS, D_MODEL), jnp.bfloat16),
            pltpu.VMEM((ROWS, D_MODEL), jnp.float32),
            pltpu.SemaphoreType.DMA((6,)),
            pltpu.SemaphoreType.DMA((6,)),
        ],
        compiler_params=pltpu.CompilerParams(collective_id=0),
    )(x2, Wq, Wo, kb, vb)

    return out.reshape(B, SQ_LOC, D_MODEL)
